# Initial kernel scaffold; baseline (speedup 1.0000x reference)
import functools

import jax
import jax.numpy as jnp
from jax import lax
from jax.experimental import pallas as pl
from jax.experimental.pallas import tpu as pltpu
from jax.experimental.pallas import tpu_sc as plsc

D = 8
GRID = 128
RANK = 32
BATCH = 16384

NC = 2
NS = 16
NW = NC * NS
BPW = BATCH // NW
CH = 32
NCH = BPW // CH


def _tt_body(idx_hbm, c0_hbm, c1_hbm, c2_hbm, c3_hbm, c4_hbm, c5_hbm,
             c6_hbm, c7_hbm, out_hbm,
             idxv, vbuf, mats, c7c, outv, sem):
    wid = lax.axis_index("s") * NC + lax.axis_index("c")
    mid_hbm = [c1_hbm, c2_hbm, c3_hbm, c4_hbm, c5_hbm, c6_hbm]

    pltpu.sync_copy(idx_hbm.at[0, wid], idxv)

    def s0_chunk(c, _):
        pltpu.async_copy(c0_hbm.at[idxv.at[c]],
                         vbuf.at[pl.ds(c * CH, CH)], sem).wait()
        return _

    lax.fori_loop(0, NCH, s0_chunk, None)

    for i in range(1, 7):
        pltpu.sync_copy(idx_hbm.at[i, wid], idxv)
        core_hbm = mid_hbm[i - 1]

        def mid_chunk(c, _, core_hbm=core_hbm):
            pltpu.async_copy(core_hbm.at[idxv.at[c]], mats, sem).wait()

            def elem(e, __):
                ge = c * CH + e
                acc0 = jnp.zeros((16,), jnp.float32)
                acc1 = jnp.zeros((16,), jnp.float32)
                for k in range(RANK):
                    vk = vbuf[ge, k]
                    acc0 = acc0 + vk * mats[e, k, pl.ds(0, 16)]
                    acc1 = acc1 + vk * mats[e, k, pl.ds(16, 16)]
                vbuf[ge, pl.ds(0, 16)] = acc0
                vbuf[ge, pl.ds(16, 16)] = acc1
                return __

            lax.fori_loop(0, CH, elem, None)
            return _

        lax.fori_loop(0, NCH, mid_chunk, None)

    pltpu.sync_copy(idx_hbm.at[7, wid], idxv)
    iota16 = lax.iota(jnp.int32, 16)

    def s7_chunk(c, _):
        pltpu.async_copy(c7_hbm.at[idxv.at[c]], c7c, sem).wait()
        for g in range(2):
            rows = iota16 + (c * CH + g * 16)
            lrows = iota16 + g * 16
            acc = jnp.zeros((16,), jnp.float32)
            for k in range(RANK):
                kvec = jnp.full((16,), k, jnp.int32)
                vk = plsc.load_gather(vbuf, [rows, kvec])
                mk = plsc.load_gather(c7c, [lrows, kvec])
                acc = acc + vk * mk
            outv[pl.ds(c * CH + g * 16, 16)] = acc
        return _

    lax.fori_loop(0, NCH, s7_chunk, None)

    pltpu.sync_copy(outv, out_hbm.at[pl.ds(wid * BPW, BPW)])


@jax.jit
def kernel(indices, core0, core1, core2, core3, core4, core5, core6, core7):
    idx4 = indices.astype(jnp.int32).T.reshape(D, NW, NCH, CH)
    c0 = core0.reshape(GRID, RANK)
    mids = [jnp.transpose(c, (1, 0, 2))
            for c in (core1, core2, core3, core4, core5, core6)]
    c7 = core7[:, :, 0].T

    mesh = plsc.VectorSubcoreMesh(core_axis_name="c", subcore_axis_name="s",
                                  num_cores=NC, num_subcores=NS)
    run = functools.partial(
        pl.kernel,
        out_type=jax.ShapeDtypeStruct((BATCH,), jnp.float32),
        mesh=mesh,
        scratch_types=[
            pltpu.VMEM((NCH, CH), jnp.int32),
            pltpu.VMEM((BPW, RANK), jnp.float32),
            pltpu.VMEM((CH, RANK, RANK), jnp.float32),
            pltpu.VMEM((CH, RANK), jnp.float32),
            pltpu.VMEM((BPW,), jnp.float32),
            pltpu.SemaphoreType.DMA,
        ],
    )(_tt_body)
    return run(idx4, c0, *mids, c7)


# speedup vs baseline: 5.3360x; 5.3360x over previous
import functools

import jax
import jax.numpy as jnp
from jax import lax
from jax.experimental import pallas as pl
from jax.experimental.pallas import tpu as pltpu
from jax.experimental.pallas import tpu_sc as plsc

D = 8
GRID = 128
RANK = 32
BATCH = 16384

NC = 2
NS = 16
NW = NC * NS
BPW = BATCH // NW
CH = 32
NCH = BPW // CH


def _tt_body(idx_hbm, c0_hbm, c1_hbm, c2_hbm, c3_hbm, c4_hbm, c5_hbm,
             c6_hbm, c7_hbm, out_hbm,
             idxv, vbuf, mats, pad0, outv, sem):
    wid = lax.axis_index("s") * NC + lax.axis_index("c")
    mid_hbm = [c1_hbm, c2_hbm, c3_hbm, c4_hbm, c5_hbm, c6_hbm]

    pltpu.sync_copy(idx_hbm.at[0, wid], idxv)

    def s0_chunk(c, _):
        pltpu.async_copy(c0_hbm.at[idxv.at[c]], pad0, sem).wait()

        def s0_elem(e, __):
            ge = c * CH + e
            vbuf[ge, pl.ds(0, 16)] = pad0[e, pl.ds(0, 16)]
            vbuf[ge, pl.ds(16, 16)] = pad0[e, pl.ds(16, 16)]
            return __

        lax.fori_loop(0, CH, s0_elem, None)
        return _

    lax.fori_loop(0, NCH, s0_chunk, None)

    for i in range(1, 7):
        pltpu.sync_copy(idx_hbm.at[i, wid], idxv)
        core_hbm = mid_hbm[i - 1]

        def mid_chunk(c, _, core_hbm=core_hbm):
            pltpu.async_copy(core_hbm.at[idxv.at[c]], mats, sem).wait()

            def elem(e, __):
                ge = c * CH + e
                r0 = vbuf[ge, pl.ds(0, 16)]
                r1 = vbuf[ge, pl.ds(16, 16)]
                acc0 = jnp.zeros((16,), jnp.float32)
                acc1 = jnp.zeros((16,), jnp.float32)
                for k in range(RANK):
                    vk = r0[k] if k < 16 else r1[k - 16]
                    acc0 = acc0 + vk * mats[e, pl.ds(k * 32, 16)]
                    acc1 = acc1 + vk * mats[e, pl.ds(k * 32 + 16, 16)]
                vbuf[ge, pl.ds(0, 16)] = acc0
                vbuf[ge, pl.ds(16, 16)] = acc1
                return __

            lax.fori_loop(0, CH, elem, None)
            return _

        lax.fori_loop(0, NCH, mid_chunk, None)

    pltpu.sync_copy(idx_hbm.at[7, wid], idxv)
    iota16 = lax.iota(jnp.int32, 16)

    def s7_chunk(c, _):
        pltpu.async_copy(c7_hbm.at[idxv.at[c]], pad0, sem).wait()

        def grp(g, __):
            yvec = jnp.zeros((16,), jnp.float32)
            for j in range(16):
                e = g * 16 + j
                ge = c * CH + e
                r0 = vbuf[ge, pl.ds(0, 16)]
                r1 = vbuf[ge, pl.ds(16, 16)]
                prod = (r0 * pad0[e, pl.ds(0, 16)]
                        + r1 * pad0[e, pl.ds(16, 16)])
                s = prod[0]
                for t in range(1, 16):
                    s = s + prod[t]
                yvec = jnp.where(iota16 == j, s, yvec)
            outv[pl.ds(c * CH + g * 16, 16)] = yvec
            return __

        lax.fori_loop(0, 2, grp, None)
        return _

    lax.fori_loop(0, NCH, s7_chunk, None)

    pltpu.sync_copy(outv, out_hbm.at[pl.ds(wid * BPW, BPW)])


@jax.jit
def kernel(indices, core0, core1, core2, core3, core4, core5, core6, core7):
    idx4 = indices.astype(jnp.int32).T.reshape(D, NW, NCH, CH)
    c0 = jnp.pad(core0.reshape(GRID, RANK), ((0, 0), (0, 96)))
    mids = [jnp.transpose(c, (1, 0, 2)).reshape(GRID, RANK * RANK)
            for c in (core1, core2, core3, core4, core5, core6)]
    c7 = jnp.pad(core7[:, :, 0].T, ((0, 0), (0, 96)))

    mesh = plsc.VectorSubcoreMesh(core_axis_name="c", subcore_axis_name="s",
                                  num_cores=NC, num_subcores=NS)
    run = functools.partial(
        pl.kernel,
        out_type=jax.ShapeDtypeStruct((BATCH,), jnp.float32),
        mesh=mesh,
        scratch_types=[
            pltpu.VMEM((NCH, CH), jnp.int32),
            pltpu.VMEM((BPW, RANK), jnp.float32),
            pltpu.VMEM((CH, RANK * RANK), jnp.float32),
            pltpu.VMEM((CH, 128), jnp.float32),
            pltpu.VMEM((BPW,), jnp.float32),
            pltpu.SemaphoreType.DMA,
        ],
    )(_tt_body)
    return run(idx4, c0, *mids, c7)


# speedup vs baseline: 6.7900x; 1.2725x over previous
import functools

import jax
import jax.numpy as jnp
from jax import lax
from jax.experimental import pallas as pl
from jax.experimental.pallas import tpu as pltpu
from jax.experimental.pallas import tpu_sc as plsc

D = 8
GRID = 128
RANK = 32
BATCH = 16384

NC = 2
NS = 16
NW = NC * NS
BPW = BATCH // NW
CH = 32
NCH = BPW // CH
NMID = 6 * NCH


def _vrow(ge):
    return ge // 4, (ge % 4) * 32


def _tt_body(idxe_hbm, idxm_hbm, c0_hbm, cm_hbm, c7_hbm, out_hbm,
             idxv, idxo, vbuf, mats0, mats1, pad0, outv, sem0, sem1):
    wid = lax.axis_index("s") * NC + lax.axis_index("c")

    pltpu.sync_copy(idxe_hbm.at[wid], idxv)
    pltpu.sync_copy(idxm_hbm.at[wid], idxo)

    def eidx(stage, c):
        return idxv.at[stage * 4 + c // 4, pl.ds((c % 4) * 32, CH)]

    def midx(t):
        return idxo.at[t // 4, pl.ds((t % 4) * 32, CH)]

    pltpu.async_copy(cm_hbm.at[midx(0)], mats0, sem0)

    def s0_chunk(c, _):
        pltpu.async_copy(c0_hbm.at[eidx(0, c)], pad0, sem1).wait()

        def s0_elem(e, __):
            r, o = _vrow(c * CH + e)
            vbuf[r, pl.ds(o, 16)] = pad0[e, pl.ds(0, 16)]
            vbuf[r, pl.ds(o + 16, 16)] = pad0[e, pl.ds(16, 16)]
            return __

        lax.fori_loop(0, CH, s0_elem, None)
        return _

    lax.fori_loop(0, NCH, s0_chunk, None)

    def compute_chunk(t, mats):
        cbase = (t % NCH) * CH

        def elem(e, __):
            r, o = _vrow(cbase + e)
            r0 = vbuf[r, pl.ds(o, 16)]
            r1 = vbuf[r, pl.ds(o + 16, 16)]
            acc0 = jnp.zeros((16,), jnp.float32)
            acc1 = jnp.zeros((16,), jnp.float32)
            for k in range(RANK):
                vk = r0[k] if k < 16 else r1[k - 16]
                acc0 = acc0 + vk * mats[e, pl.ds(k * 32, 16)]
                acc1 = acc1 + vk * mats[e, pl.ds(k * 32 + 16, 16)]
            vbuf[r, pl.ds(o, 16)] = acc0
            vbuf[r, pl.ds(o + 16, 16)] = acc1
            return __

        lax.fori_loop(0, CH, elem, None)

    def mid_pair(t2, _):
        t = 2 * t2
        pltpu.async_copy(cm_hbm.at[midx(t + 1)], mats1, sem1)
        pltpu.make_async_copy(cm_hbm.at[midx(t)], mats0, sem0).wait()
        compute_chunk(t, mats0)
        tnx = jnp.minimum(t + 2, NMID - 1)
        pltpu.async_copy(cm_hbm.at[midx(tnx)], mats0, sem0)
        pltpu.make_async_copy(cm_hbm.at[midx(t + 1)], mats1, sem1).wait()
        compute_chunk(t + 1, mats1)
        return _

    lax.fori_loop(0, NMID // 2, mid_pair, None)
    pltpu.make_async_copy(cm_hbm.at[midx(0)], mats0, sem0).wait()

    iota16 = lax.iota(jnp.int32, 16)

    def s7_chunk(c, _):
        pltpu.async_copy(c7_hbm.at[eidx(1, c)], pad0, sem1).wait()

        def grp(g, __):
            yvec = jnp.zeros((16,), jnp.float32)
            for j in range(16):
                e = g * 16 + j
                r, o = _vrow(c * CH + e)
                r0 = vbuf[r, pl.ds(o, 16)]
                r1 = vbuf[r, pl.ds(o + 16, 16)]
                prod = (r0 * pad0[e, pl.ds(0, 16)]
                        + r1 * pad0[e, pl.ds(16, 16)])
                s = prod[0]
                for u in range(1, 16):
                    s = s + prod[u]
                yvec = jnp.where(iota16 == j, s, yvec)
            outv[pl.ds(c * CH + g * 16, 16)] = yvec
            return __

        lax.fori_loop(0, 2, grp, None)
        return _

    lax.fori_loop(0, NCH, s7_chunk, None)

    pltpu.sync_copy(outv, out_hbm.at[pl.ds(wid * BPW, BPW)])


@jax.jit
def kernel(indices, core0, core1, core2, core3, core4, core5, core6, core7):
    idx = indices.astype(jnp.int32)
    idxe = jnp.stack([idx[:, 0], idx[:, 7]], 0)
    idxe = idxe.reshape(2, NW, BPW).transpose(1, 0, 2).reshape(NW, 8, 128)
    idxm = (idx[:, 1:7] + jnp.arange(6, dtype=jnp.int32) * GRID).T
    idxm = idxm.reshape(6, NW, BPW).transpose(1, 0, 2).reshape(NW, 24, 128)
    c0 = jnp.pad(core0.reshape(GRID, RANK), ((0, 0), (0, 96)))
    cm = jnp.stack([jnp.transpose(c, (1, 0, 2))
                    for c in (core1, core2, core3, core4, core5, core6)])
    cm = cm.reshape(6 * GRID, RANK * RANK)
    c7 = jnp.pad(core7[:, :, 0].T, ((0, 0), (0, 96)))

    mesh = plsc.VectorSubcoreMesh(core_axis_name="c", subcore_axis_name="s",
                                  num_cores=NC, num_subcores=NS)
    run = functools.partial(
        pl.kernel,
        out_type=jax.ShapeDtypeStruct((BATCH,), jnp.float32),
        mesh=mesh,
        scratch_types=[
            pltpu.VMEM((8, 128), jnp.int32),
            pltpu.VMEM((24, 128), jnp.int32),
            pltpu.VMEM((128, 128), jnp.float32),
            pltpu.VMEM((CH, RANK * RANK), jnp.float32),
            pltpu.VMEM((CH, RANK * RANK), jnp.float32),
            pltpu.VMEM((CH, 128), jnp.float32),
            pltpu.VMEM((BPW,), jnp.float32),
            pltpu.SemaphoreType.DMA,
            pltpu.SemaphoreType.DMA,
        ],
    )(_tt_body)
    return run(idxe, idxm, c0, cm, c7)


# speedup vs baseline: 8.5049x; 1.2526x over previous
import functools

import jax
import jax.numpy as jnp
from jax import lax
from jax.experimental import pallas as pl
from jax.experimental.pallas import tpu as pltpu
from jax.experimental.pallas import tpu_sc as plsc

D = 8
GRID = 128
RANK = 32
BATCH = 16384

NC = 2
NS = 16
NW = NC * NS
BPW = BATCH // NW
CH = 32
NCH = BPW // CH
NMID = 6 * NCH


def _vrow(ge):
    return ge // 4, (ge % 4) * 32


def _tt_body(idxe_hbm, idxm_hbm, c0_hbm, cm_hbm, c7_hbm, out_hbm,
             idxv, idxo, vbuf, mats0, mats1, pad0, outv, sem0, sem1):
    wid = lax.axis_index("s") * NC + lax.axis_index("c")

    pltpu.sync_copy(idxe_hbm.at[wid], idxv)
    pltpu.sync_copy(idxm_hbm.at[wid], idxo)

    def eidx(stage, c):
        return idxv.at[stage * 4 + c // 4, pl.ds((c % 4) * 32, CH)]

    def midx(t):
        return idxo.at[t // 4, pl.ds((t % 4) * 32, CH)]

    pltpu.async_copy(cm_hbm.at[midx(0)], mats0, sem0)

    def s0_chunk(c, _):
        pltpu.async_copy(c0_hbm.at[eidx(0, c)], pad0, sem1).wait()

        def s0_elem(e, __):
            r, o = _vrow(c * CH + e)
            vbuf[r, pl.ds(o, 16)] = pad0[e, pl.ds(0, 16)]
            vbuf[r, pl.ds(o + 16, 16)] = pad0[e, pl.ds(16, 16)]
            return __

        lax.fori_loop(0, CH, s0_elem, None)
        return _

    lax.fori_loop(0, NCH, s0_chunk, None)

    def compute_chunk(t, mats):
        cbase = (t % NCH) * CH

        def elem(e, __):
            r, o = _vrow(cbase + e)
            r0 = vbuf[r, pl.ds(o, 16)]
            r1 = vbuf[r, pl.ds(o + 16, 16)]
            acc0 = jnp.zeros((16,), jnp.float32)
            acc1 = jnp.zeros((16,), jnp.float32)
            himask = jnp.full((16,), -65536, jnp.int32)
            for k in range(RANK):
                vk = r0[k] if k < 16 else r1[k - 16]
                mw = mats[e, pl.ds(k * 16, 16)]
                ma = plsc.bitcast(mw << 16, jnp.float32)
                mb = plsc.bitcast(mw & himask, jnp.float32)
                acc0 = acc0 + vk * ma
                acc1 = acc1 + vk * mb
            vbuf[r, pl.ds(o, 16)] = acc0
            vbuf[r, pl.ds(o + 16, 16)] = acc1
            return __

        lax.fori_loop(0, CH, elem, None)

    def mid_pair(t2, _):
        t = 2 * t2
        pltpu.async_copy(cm_hbm.at[midx(t + 1)], mats1, sem1)
        pltpu.make_async_copy(cm_hbm.at[midx(t)], mats0, sem0).wait()
        compute_chunk(t, mats0)
        tnx = jnp.minimum(t + 2, NMID - 1)
        pltpu.async_copy(cm_hbm.at[midx(tnx)], mats0, sem0)
        pltpu.make_async_copy(cm_hbm.at[midx(t + 1)], mats1, sem1).wait()
        compute_chunk(t + 1, mats1)
        return _

    lax.fori_loop(0, NMID // 2, mid_pair, None)
    pltpu.make_async_copy(cm_hbm.at[midx(0)], mats0, sem0).wait()

    iota16 = lax.iota(jnp.int32, 16)

    def s7_chunk(c, _):
        pltpu.async_copy(c7_hbm.at[eidx(1, c)], pad0, sem1).wait()

        def grp(g, __):
            yvec = jnp.zeros((16,), jnp.float32)
            for j in range(16):
                e = g * 16 + j
                r, o = _vrow(c * CH + e)
                r0 = vbuf[r, pl.ds(o, 16)]
                r1 = vbuf[r, pl.ds(o + 16, 16)]
                prod = (r0 * pad0[e, pl.ds(0, 16)]
                        + r1 * pad0[e, pl.ds(16, 16)])
                s = prod[0]
                for u in range(1, 16):
                    s = s + prod[u]
                yvec = jnp.where(iota16 == j, s, yvec)
            outv[pl.ds(c * CH + g * 16, 16)] = yvec
            return __

        lax.fori_loop(0, 2, grp, None)
        return _

    lax.fori_loop(0, NCH, s7_chunk, None)

    pltpu.sync_copy(outv, out_hbm.at[pl.ds(wid * BPW, BPW)])


@jax.jit
def kernel(indices, core0, core1, core2, core3, core4, core5, core6, core7):
    idx = indices.astype(jnp.int32)
    idxe = jnp.stack([idx[:, 0], idx[:, 7]], 0)
    idxe = idxe.reshape(2, NW, BPW).transpose(1, 0, 2).reshape(NW, 8, 128)
    idxm = (idx[:, 1:7] + jnp.arange(6, dtype=jnp.int32) * GRID).T
    idxm = idxm.reshape(6, NW, BPW).transpose(1, 0, 2).reshape(NW, 24, 128)
    c0 = jnp.pad(core0.reshape(GRID, RANK), ((0, 0), (0, 96)))
    cm = jnp.stack([jnp.transpose(c, (1, 0, 2))
                    for c in (core1, core2, core3, core4, core5, core6)])
    perm = jnp.stack([jnp.arange(16), jnp.arange(16, 32)], 1).reshape(-1)
    cm = cm.reshape(6, GRID, RANK, RANK)[:, :, :, perm].astype(jnp.bfloat16)
    cm = jax.lax.bitcast_convert_type(
        cm.reshape(6 * GRID, RANK * 16, 2), jnp.int32).reshape(6 * GRID, 512)
    c7 = jnp.pad(core7[:, :, 0].T, ((0, 0), (0, 96)))

    mesh = plsc.VectorSubcoreMesh(core_axis_name="c", subcore_axis_name="s",
                                  num_cores=NC, num_subcores=NS)
    run = functools.partial(
        pl.kernel,
        out_type=jax.ShapeDtypeStruct((BATCH,), jnp.float32),
        mesh=mesh,
        compiler_params=pltpu.CompilerParams(needs_layout_passes=False),
        scratch_types=[
            pltpu.VMEM((8, 128), jnp.int32),
            pltpu.VMEM((24, 128), jnp.int32),
            pltpu.VMEM((128, 128), jnp.float32),
            pltpu.VMEM((CH, 512), jnp.int32),
            pltpu.VMEM((CH, 512), jnp.int32),
            pltpu.VMEM((CH, 128), jnp.float32),
            pltpu.VMEM((BPW,), jnp.float32),
            pltpu.SemaphoreType.DMA,
            pltpu.SemaphoreType.DMA,
        ],
    )(_tt_body)
    return run(idxe, idxm, c0, cm, c7)


# speedup vs baseline: 8.5289x; 1.0028x over previous
import functools

import jax
import jax.numpy as jnp
from jax import lax
from jax.experimental import pallas as pl
from jax.experimental.pallas import tpu as pltpu
from jax.experimental.pallas import tpu_sc as plsc

D = 8
GRID = 128
RANK = 32
BATCH = 16384

NC = 2
NS = 16
NW = NC * NS
BPW = BATCH // NW
CH = 32
NCH = BPW // CH
NMID = 6 * NCH


def _vrow(ge):
    return ge // 4, (ge % 4) * 32


def _tt_body(idxe_hbm, idxm_hbm, c0_hbm, cm_hbm, c7_hbm, out_hbm,
             idxv, idxo, vbuf, mats0, mats1, pad0, outv, sem0, sem1):
    wid = lax.axis_index("s") * NC + lax.axis_index("c")

    pltpu.sync_copy(idxe_hbm.at[wid], idxv)
    pltpu.sync_copy(idxm_hbm.at[wid], idxo)

    def eidx(stage, c):
        return idxv.at[stage * 4 + c // 4, pl.ds((c % 4) * 32, CH)]

    def midx(t):
        return idxo.at[t // 4, pl.ds((t % 4) * 32, CH)]

    pltpu.async_copy(cm_hbm.at[midx(0)], mats0, sem0)

    def s0_chunk(c, _):
        pltpu.async_copy(c0_hbm.at[eidx(0, c)], pad0, sem1).wait()

        def s0_elem(e, __):
            r, o = _vrow(c * CH + e)
            vbuf[r, pl.ds(o, 16)] = pad0[e, pl.ds(0, 16)]
            vbuf[r, pl.ds(o + 16, 16)] = pad0[e, pl.ds(16, 16)]
            return __

        lax.fori_loop(0, CH, s0_elem, None)
        return _

    lax.fori_loop(0, NCH, s0_chunk, None)

    def compute_chunk(t, mats):
        cbase = (t % NCH) * CH
        himask = jnp.full((16,), -65536, jnp.int32)

        def elem(ep, __):
            for sub in range(2):
                e = ep * 2 + sub
                r, o = _vrow(cbase + e)
                r0 = vbuf[r, pl.ds(o, 16)]
                r1 = vbuf[r, pl.ds(o + 16, 16)]
                acc0a = jnp.zeros((16,), jnp.float32)
                acc0b = jnp.zeros((16,), jnp.float32)
                acc1a = jnp.zeros((16,), jnp.float32)
                acc1b = jnp.zeros((16,), jnp.float32)
                for k in range(RANK):
                    vk = r0[k] if k < 16 else r1[k - 16]
                    mw = mats[e, pl.ds(k * 16, 16)]
                    ma = plsc.bitcast(mw << 16, jnp.float32)
                    mb = plsc.bitcast(mw & himask, jnp.float32)
                    if k % 2 == 0:
                        acc0a = acc0a + vk * ma
                        acc1a = acc1a + vk * mb
                    else:
                        acc0b = acc0b + vk * ma
                        acc1b = acc1b + vk * mb
                vbuf[r, pl.ds(o, 16)] = acc0a + acc0b
                vbuf[r, pl.ds(o + 16, 16)] = acc1a + acc1b
            return __

        lax.fori_loop(0, CH // 2, elem, None)

    def mid_pair(t2, _):
        t = 2 * t2
        pltpu.async_copy(cm_hbm.at[midx(t + 1)], mats1, sem1)
        pltpu.make_async_copy(cm_hbm.at[midx(t)], mats0, sem0).wait()
        compute_chunk(t, mats0)
        tnx = jnp.minimum(t + 2, NMID - 1)
        pltpu.async_copy(cm_hbm.at[midx(tnx)], mats0, sem0)
        pltpu.make_async_copy(cm_hbm.at[midx(t + 1)], mats1, sem1).wait()
        compute_chunk(t + 1, mats1)
        return _

    lax.fori_loop(0, NMID // 2, mid_pair, None)
    pltpu.make_async_copy(cm_hbm.at[midx(0)], mats0, sem0).wait()

    iota16 = lax.iota(jnp.int32, 16)

    def s7_chunk(c, _):
        pltpu.async_copy(c7_hbm.at[eidx(1, c)], pad0, sem1).wait()

        def grp(g, __):
            yvec = jnp.zeros((16,), jnp.float32)
            for j in range(16):
                e = g * 16 + j
                r, o = _vrow(c * CH + e)
                r0 = vbuf[r, pl.ds(o, 16)]
                r1 = vbuf[r, pl.ds(o + 16, 16)]
                prod = (r0 * pad0[e, pl.ds(0, 16)]
                        + r1 * pad0[e, pl.ds(16, 16)])
                s = prod[0]
                for u in range(1, 16):
                    s = s + prod[u]
                yvec = jnp.where(iota16 == j, s, yvec)
            outv[pl.ds(c * CH + g * 16, 16)] = yvec
            return __

        lax.fori_loop(0, 2, grp, None)
        return _

    lax.fori_loop(0, NCH, s7_chunk, None)

    pltpu.sync_copy(outv, out_hbm.at[pl.ds(wid * BPW, BPW)])


@jax.jit
def kernel(indices, core0, core1, core2, core3, core4, core5, core6, core7):
    idx = indices.astype(jnp.int32)
    idxe = jnp.stack([idx[:, 0], idx[:, 7]], 0)
    idxe = idxe.reshape(2, NW, BPW).transpose(1, 0, 2).reshape(NW, 8, 128)
    idxm = (idx[:, 1:7] + jnp.arange(6, dtype=jnp.int32) * GRID).T
    idxm = idxm.reshape(6, NW, BPW).transpose(1, 0, 2).reshape(NW, 24, 128)
    c0 = jnp.pad(core0.reshape(GRID, RANK), ((0, 0), (0, 96)))
    cm = jnp.stack([jnp.transpose(c, (1, 0, 2))
                    for c in (core1, core2, core3, core4, core5, core6)])
    perm = jnp.stack([jnp.arange(16), jnp.arange(16, 32)], 1).reshape(-1)
    cm = cm.reshape(6, GRID, RANK, RANK)[:, :, :, perm].astype(jnp.bfloat16)
    cm = jax.lax.bitcast_convert_type(
        cm.reshape(6 * GRID, RANK * 16, 2), jnp.int32).reshape(6 * GRID, 512)
    c7 = jnp.pad(core7[:, :, 0].T, ((0, 0), (0, 96)))

    mesh = plsc.VectorSubcoreMesh(core_axis_name="c", subcore_axis_name="s",
                                  num_cores=NC, num_subcores=NS)
    run = functools.partial(
        pl.kernel,
        out_type=jax.ShapeDtypeStruct((BATCH,), jnp.float32),
        mesh=mesh,
        compiler_params=pltpu.CompilerParams(needs_layout_passes=False),
        scratch_types=[
            pltpu.VMEM((8, 128), jnp.int32),
            pltpu.VMEM((24, 128), jnp.int32),
            pltpu.VMEM((128, 128), jnp.float32),
            pltpu.VMEM((CH, 512), jnp.int32),
            pltpu.VMEM((CH, 512), jnp.int32),
            pltpu.VMEM((CH, 128), jnp.float32),
            pltpu.VMEM((BPW,), jnp.float32),
            pltpu.SemaphoreType.DMA,
            pltpu.SemaphoreType.DMA,
        ],
    )(_tt_body)
    return run(idxe, idxm, c0, cm, c7)
